# Initial kernel scaffold; baseline (speedup 1.0000x reference)
import jax
import jax.numpy as jnp
from jax.experimental import pallas as pl


def kernel(u_cat, u_num, T_light, T_hum, T_care, T_size, T_climate, T_water, T_care_freq, W1, b1, W2, b2, W3, b3):
    raise NotImplementedError("write your pallas kernel here")


# speedup vs baseline: 7.8140x; 7.8140x over previous
import functools

import jax
import jax.numpy as jnp
from jax.experimental import pallas as pl

_VOCABS = (6, 4, 4, 4, 6, 4, 4)
_OFF = (0, 6, 10, 14, 18, 24, 28)
_EMBED = 8
_B = 16384
_BS = 2048


def _body(uc_ref, un_ref, mext_ref, w1_ref, b1_ref, w2_ref, b2_ref, w3_ref,
          b3_ref, out_ref):
    bs = uc_ref.shape[0]
    uc = uc_ref[...]
    un = un_ref[...]
    lane = jax.lax.broadcasted_iota(jnp.int32, (bs, 40), 1)
    a = jnp.zeros((bs, 40), jnp.float32)
    for i in range(7):
        a = a + (lane == uc[:, i:i + 1] + _OFF[i]).astype(jnp.float32)
    a = jnp.where(lane == 32, un[:, 0:1], a)
    a = jnp.where(lane == 33, un[:, 1:2], a)
    g = jnp.dot(mext_ref[...], w1_ref[...], preferred_element_type=jnp.float32)
    h = jnp.dot(a, g, preferred_element_type=jnp.float32) + b1_ref[...]
    h = jnp.maximum(h, 0.0)
    h = jnp.dot(h, w2_ref[...], preferred_element_type=jnp.float32) + b2_ref[...]
    h = jnp.maximum(h, 0.0)
    out_ref[...] = (jnp.dot(h, w3_ref[...], preferred_element_type=jnp.float32)
                    + b3_ref[...])


@functools.partial(jax.jit, static_argnames=("interpret",))
def kernel(u_cat, u_num, T_light, T_hum, T_care, T_size, T_climate, T_water,
           T_care_freq, W1, b1, W2, b2, W3, b3, interpret=False):
    tables = [T_light, T_hum, T_care, T_size, T_climate, T_water, T_care_freq]
    mext = jnp.zeros((40, 64), jnp.float32)
    for i, (t, o) in enumerate(zip(tables, _OFF)):
        mext = mext.at[o:o + _VOCABS[i], 8 * i:8 * i + _EMBED].set(t)
    mext = mext.at[32, 56].set(1.0).at[33, 57].set(1.0)
    w1p = jnp.zeros((64, 128), jnp.float32).at[:58].set(W1)

    uc = u_cat.astype(jnp.int32)
    grid = (_B // _BS,)
    out = pl.pallas_call(
        _body,
        grid=grid,
        in_specs=[
            pl.BlockSpec((_BS, 7), lambda i: (i, 0)),
            pl.BlockSpec((_BS, 2), lambda i: (i, 0)),
            pl.BlockSpec((40, 64), lambda i: (0, 0)),
            pl.BlockSpec((64, 128), lambda i: (0, 0)),
            pl.BlockSpec((1, 128), lambda i: (0, 0)),
            pl.BlockSpec((128, 128), lambda i: (0, 0)),
            pl.BlockSpec((1, 128), lambda i: (0, 0)),
            pl.BlockSpec((128, 64), lambda i: (0, 0)),
            pl.BlockSpec((1, 64), lambda i: (0, 0)),
        ],
        out_specs=pl.BlockSpec((_BS, 64), lambda i: (i, 0)),
        out_shape=jax.ShapeDtypeStruct((_B, 64), jnp.float32),
        interpret=interpret,
    )(uc, u_num, mext, w1p, b1.reshape(1, 128), W2, b2.reshape(1, 128), W3,
      b3.reshape(1, 64))
    return out
